# Initial kernel scaffold; baseline (speedup 1.0000x reference)
import jax
import jax.numpy as jnp
from jax.experimental import pallas as pl


def kernel(x, edge_index, W1, b1, W2, b2):
    raise NotImplementedError("write your pallas kernel here")


# speedup vs baseline: 13.2776x; 13.2776x over previous
import functools

import jax
import jax.numpy as jnp
from jax import lax
from jax.experimental import pallas as pl
from jax.experimental.pallas import tpu as pltpu
from jax.experimental.pallas import tpu_sc as plsc

N = 10000
E = 320000
F = 128
HID = 256

NC = 2
NS = 16
NW = NC * NS
EPT = E // NW
CH = 80
NCHUNK = EPT // CH
RSTEP = 624
RLEN = 640

def _mesh():
    return plsc.VectorSubcoreMesh(
        core_axis_name="c", subcore_axis_name="s", num_cores=NC, num_subcores=NS
    )


def _deg_body(dst_hbm, zeros16_hbm, hist_hbm, ones_v, idx_v, hist_sh, sem):
    c = lax.axis_index("c")
    s = lax.axis_index("s")
    for i in range(CH):
        ones_v[i] = jnp.ones((16,), jnp.float32)
    pltpu.sync_copy(
        zeros16_hbm.at[pl.ds(s * RSTEP, RLEN)], hist_sh.at[pl.ds(s * RSTEP, RLEN)]
    )
    plsc.subcore_barrier()

    ebase = (c * NS + s) * EPT

    def step(i, carry):
        pltpu.sync_copy(dst_hbm.at[pl.ds(ebase + i * CH, CH)], idx_v)
        pltpu.sync_copy(ones_v, hist_sh.at[idx_v], add=True)
        return carry

    lax.fori_loop(0, NCHUNK, step, 0)
    plsc.subcore_barrier()
    pltpu.sync_copy(
        hist_sh.at[pl.ds(s * RSTEP, RLEN)], hist_hbm.at[c, pl.ds(s * RSTEP, RLEN)]
    )


@functools.cache
def _deg_call():
    return pl.kernel(
        _deg_body,
        out_type=jax.ShapeDtypeStruct((NC, N, 16), jnp.float32),
        mesh=_mesh(),
        scratch_types=[
            pltpu.VMEM((CH, 16), jnp.float32),
            pltpu.VMEM((CH,), jnp.int32),
            pltpu.VMEM_SHARED((N, 16), jnp.float32),
            pltpu.SemaphoreType.DMA,
        ],
    )


def _agg_body(z_hbm, src_hbm, dst_hbm, zeros_hbm, out_hbm,
              sidx_v, didx_v, rows_v, acc_sh, sem):
    c = lax.axis_index("c")
    s = lax.axis_index("s")
    pltpu.sync_copy(
        zeros_hbm.at[pl.ds(s * RSTEP, RLEN)], acc_sh.at[pl.ds(s * RSTEP, RLEN)]
    )
    plsc.subcore_barrier()

    ebase = (c * NS + s) * EPT

    def step(i, carry):
        base = ebase + i * CH
        pltpu.sync_copy(src_hbm.at[pl.ds(base, CH)], sidx_v)
        pltpu.sync_copy(dst_hbm.at[pl.ds(base, CH)], didx_v)
        pltpu.async_copy(z_hbm.at[sidx_v], rows_v, sem).wait()
        pltpu.sync_copy(rows_v, acc_sh.at[didx_v], add=True)
        return carry

    lax.fori_loop(0, NCHUNK, step, 0)
    plsc.subcore_barrier()
    pltpu.sync_copy(
        acc_sh.at[pl.ds(s * RSTEP, RLEN)], out_hbm.at[c, pl.ds(s * RSTEP, RLEN)]
    )


@functools.cache
def _agg_call():
    return pl.kernel(
        _agg_body,
        out_type=jax.ShapeDtypeStruct((NC, N, F), jnp.float32),
        mesh=_mesh(),
        scratch_types=[
            pltpu.VMEM((CH,), jnp.int32),
            pltpu.VMEM((CH,), jnp.int32),
            pltpu.VMEM((CH, F), jnp.float32),
            pltpu.VMEM_SHARED((N, F), jnp.float32),
            pltpu.SemaphoreType.DMA,
        ],
    )


BLK = 200


def _d_from_hist(hist_blk):
    deg = hist_blk[0, :, 0] + hist_blk[1, :, 0] + 1.0
    return lax.rsqrt(deg)


def _tc1_body(hist_ref, x_ref, z1_ref):
    d = _d_from_hist(hist_ref[...])
    z1_ref[...] = x_ref[...] * d[:, None]


def _tc2_body(hist_ref, p_ref, z1_ref, w1_ref, b1_ref, w2_ref, z2_ref):
    d = _d_from_hist(hist_ref[...])
    p = p_ref[...]
    agg = (p[0] + p[1] + z1_ref[...]) * d[:, None]
    h = jnp.dot(agg, w1_ref[...], preferred_element_type=jnp.float32)
    h = jnp.maximum(h + b1_ref[...], 0.0)
    g = jnp.dot(h, w2_ref[...], preferred_element_type=jnp.float32)
    z2_ref[...] = g * d[:, None]


def _tc3_body(hist_ref, q_ref, z2_ref, b2_ref, out_ref):
    d = _d_from_hist(hist_ref[...])
    q = q_ref[...]
    out_ref[...] = (q[0] + q[1] + z2_ref[...]) * d[:, None] + b2_ref[...]


_hist_spec = pl.BlockSpec((NC, BLK, 16), lambda i: (0, i, 0))
_row_spec = pl.BlockSpec((BLK, F), lambda i: (i, 0))
_pair_spec = pl.BlockSpec((NC, BLK, F), lambda i: (0, i, 0))


def _tc1(hist, x):
    return pl.pallas_call(
        _tc1_body,
        grid=(N // BLK,),
        in_specs=[_hist_spec, _row_spec],
        out_specs=_row_spec,
        out_shape=jax.ShapeDtypeStruct((N, F), jnp.float32),
    )(hist, x)


def _tc2(hist, p, z1, w1, b1, w2):
    return pl.pallas_call(
        _tc2_body,
        grid=(N // BLK,),
        in_specs=[
            _hist_spec,
            _pair_spec,
            _row_spec,
            pl.BlockSpec((F, HID), lambda i: (0, 0)),
            pl.BlockSpec((1, HID), lambda i: (0, 0)),
            pl.BlockSpec((HID, F), lambda i: (0, 0)),
        ],
        out_specs=_row_spec,
        out_shape=jax.ShapeDtypeStruct((N, F), jnp.float32),
    )(hist, p, z1, w1, b1, w2)


def _tc3(hist, q, z2, b2):
    return pl.pallas_call(
        _tc3_body,
        grid=(N // BLK,),
        in_specs=[
            _hist_spec,
            _pair_spec,
            _row_spec,
            pl.BlockSpec((1, F), lambda i: (0, 0)),
        ],
        out_specs=_row_spec,
        out_shape=jax.ShapeDtypeStruct((N, F), jnp.float32),
    )(hist, q, z2, b2)


def kernel(x, edge_index, W1, b1, W2, b2):
    x = x.astype(jnp.float32)
    src = edge_index[0].astype(jnp.int32)
    dst = edge_index[1].astype(jnp.int32)
    zeros16 = jnp.zeros((N, 16), jnp.float32)
    zeros128 = jnp.zeros((N, F), jnp.float32)
    b1r = b1.reshape(1, HID).astype(jnp.float32)
    b2r = b2.reshape(1, F).astype(jnp.float32)

    hist = _deg_call()(dst, zeros16)
    z1 = _tc1(hist, x)
    p = _agg_call()(z1, src, dst, zeros128)
    z2 = _tc2(hist, p, z1, W1.astype(jnp.float32), b1r, W2.astype(jnp.float32))
    q = _agg_call()(z2, src, dst, zeros128)
    return _tc3(hist, q, z2, b2r)


# speedup vs baseline: 24.2218x; 1.8243x over previous
import functools

import jax
import jax.numpy as jnp
from jax import lax
from jax.experimental import pallas as pl
from jax.experimental.pallas import tpu as pltpu
from jax.experimental.pallas import tpu_sc as plsc

N = 10000
E = 320000
F = 128
HID = 256

NC = 2
NS = 16
NW = NC * NS
EPT = E // NW
CH = 80
NCHUNK = EPT // CH

RSTEP = 624
RLEN = 640


def _mesh():
    return plsc.VectorSubcoreMesh(
        core_axis_name="c", subcore_axis_name="s", num_cores=NC, num_subcores=NS
    )


def _deg_body(dst_hbm, zeros16_hbm, hist_hbm, ones_v, didx0, didx1,
              hist_sh, isem0, isem1):
    c = lax.axis_index("c")
    s = lax.axis_index("s")
    t = c * NS + s
    for i in range(CH):
        ones_v[i] = jnp.ones((16,), jnp.float32)
    pltpu.sync_copy(
        zeros16_hbm.at[pl.ds(s * RSTEP, RLEN)], hist_sh.at[pl.ds(s * RSTEP, RLEN)]
    )
    plsc.subcore_barrier()

    ebase = t * EPT

    def step(g, carry):
        pltpu.sync_copy(dst_hbm.at[pl.ds(ebase + g * CH, CH)], didx0)
        pltpu.sync_copy(ones_v, hist_sh.at[didx0], add=True)
        return carry

    lax.fori_loop(0, NCHUNK, step, 0)

    plsc.subcore_barrier()
    pltpu.sync_copy(
        hist_sh.at[pl.ds(s * RSTEP, RLEN)], hist_hbm.at[c, pl.ds(s * RSTEP, RLEN)]
    )


@functools.cache
def _deg_call():
    return pl.kernel(
        _deg_body,
        out_type=jax.ShapeDtypeStruct((NC, N, 16), jnp.float32),
        mesh=_mesh(),
        scratch_types=[
            pltpu.VMEM((CH, 16), jnp.float32),
            pltpu.VMEM((CH,), jnp.int32),
            pltpu.VMEM((CH,), jnp.int32),
            pltpu.VMEM_SHARED((N, 16), jnp.float32),
            pltpu.SemaphoreType.DMA,
            pltpu.SemaphoreType.DMA,
        ],
    )


def _agg_body(z_hbm, src_hbm, dst_hbm, zeros_hbm, out_hbm,
              sidx_v, didx_v, rows0, rows1, acc_sh, gsem0, gsem1):
    c = lax.axis_index("c")
    s = lax.axis_index("s")
    t = c * NS + s
    pltpu.sync_copy(
        zeros_hbm.at[pl.ds(s * RSTEP, RLEN)], acc_sh.at[pl.ds(s * RSTEP, RLEN)]
    )
    pltpu.sync_copy(src_hbm.at[pl.ds(t * EPT, EPT)], sidx_v)
    pltpu.sync_copy(dst_hbm.at[t], didx_v)
    plsc.subcore_barrier()

    rows = (rows0, rows1)
    gsem = (gsem0, gsem1)

    def _wait_gather(b):
        pltpu.make_async_copy(z_hbm.at[pl.ds(0, CH)], rows[b], gsem[b]).wait()

    def _sidx(g):
        return sidx_v.at[pl.ds(g * CH, CH)]

    pltpu.async_copy(z_hbm.at[_sidx(0)], rows0, gsem0)
    pltpu.async_copy(z_hbm.at[_sidx(1)], rows1, gsem1)

    def step(k, carry):
        for b in range(2):
            g = 2 * k + b
            _wait_gather(b)
            pltpu.sync_copy(rows[b], acc_sh.at[didx_v.at[g]], add=True)
            pltpu.async_copy(z_hbm.at[_sidx(g + 2)], rows[b], gsem[b])
        return carry

    lax.fori_loop(0, NCHUNK // 2 - 1, step, 0)
    _wait_gather(0)
    pltpu.sync_copy(rows0, acc_sh.at[didx_v.at[NCHUNK - 3]], add=True)
    pltpu.async_copy(z_hbm.at[_sidx(NCHUNK - 1)], rows0, gsem0)
    _wait_gather(1)
    pltpu.sync_copy(rows1, acc_sh.at[didx_v.at[NCHUNK - 2]], add=True)
    _wait_gather(0)
    pltpu.sync_copy(rows0, acc_sh.at[didx_v.at[NCHUNK - 1]], add=True)

    plsc.subcore_barrier()
    pltpu.sync_copy(
        acc_sh.at[pl.ds(s * RSTEP, RLEN)], out_hbm.at[c, pl.ds(s * RSTEP, RLEN)]
    )


@functools.cache
def _agg_call():
    return pl.kernel(
        _agg_body,
        out_type=jax.ShapeDtypeStruct((NC, N, F), jnp.float32),
        mesh=_mesh(),
        scratch_types=[
            pltpu.VMEM((EPT,), jnp.int32),
            pltpu.VMEM((NCHUNK, CH), jnp.int32),
            pltpu.VMEM((CH, F), jnp.float32),
            pltpu.VMEM((CH, F), jnp.float32),
            pltpu.VMEM_SHARED((N, F), jnp.float32),
            pltpu.SemaphoreType.DMA,
            pltpu.SemaphoreType.DMA,
        ],
    )


BLK = 200


def _d_from_hist(hist_blk):
    deg = hist_blk[0, :, 0] + hist_blk[1, :, 0] + 1.0
    return lax.rsqrt(deg)


def _tc1_body(hist_ref, x_ref, z1_ref):
    d = _d_from_hist(hist_ref[...])
    z1_ref[...] = x_ref[...] * d[:, None]


def _tc2_body(hist_ref, p_ref, z1_ref, w1_ref, b1_ref, w2_ref, z2_ref):
    d = _d_from_hist(hist_ref[...])
    p = p_ref[...]
    agg = (p[0] + p[1] + z1_ref[...]) * d[:, None]
    h = jnp.dot(agg, w1_ref[...], preferred_element_type=jnp.float32)
    h = jnp.maximum(h + b1_ref[...], 0.0)
    g = jnp.dot(h, w2_ref[...], preferred_element_type=jnp.float32)
    z2_ref[...] = g * d[:, None]


def _tc3_body(hist_ref, q_ref, z2_ref, b2_ref, out_ref):
    d = _d_from_hist(hist_ref[...])
    q = q_ref[...]
    out_ref[...] = (q[0] + q[1] + z2_ref[...]) * d[:, None] + b2_ref[...]


_hist_spec = pl.BlockSpec((NC, BLK, 16), lambda i: (0, i, 0))
_row_spec = pl.BlockSpec((BLK, F), lambda i: (i, 0))
_pair_spec = pl.BlockSpec((NC, BLK, F), lambda i: (0, i, 0))


def _tc1(hist, x):
    return pl.pallas_call(
        _tc1_body,
        grid=(N // BLK,),
        in_specs=[_hist_spec, _row_spec],
        out_specs=_row_spec,
        out_shape=jax.ShapeDtypeStruct((N, F), jnp.float32),
    )(hist, x)


def _tc2(hist, p, z1, w1, b1, w2):
    return pl.pallas_call(
        _tc2_body,
        grid=(N // BLK,),
        in_specs=[
            _hist_spec,
            _pair_spec,
            _row_spec,
            pl.BlockSpec((F, HID), lambda i: (0, 0)),
            pl.BlockSpec((1, HID), lambda i: (0, 0)),
            pl.BlockSpec((HID, F), lambda i: (0, 0)),
        ],
        out_specs=_row_spec,
        out_shape=jax.ShapeDtypeStruct((N, F), jnp.float32),
    )(hist, p, z1, w1, b1, w2)


def _tc3(hist, q, z2, b2):
    return pl.pallas_call(
        _tc3_body,
        grid=(N // BLK,),
        in_specs=[
            _hist_spec,
            _pair_spec,
            _row_spec,
            pl.BlockSpec((1, F), lambda i: (0, 0)),
        ],
        out_specs=_row_spec,
        out_shape=jax.ShapeDtypeStruct((N, F), jnp.float32),
    )(hist, q, z2, b2)


def kernel(x, edge_index, W1, b1, W2, b2):
    x = x.astype(jnp.float32)
    src = edge_index[0].astype(jnp.int32)
    dstf = edge_index[1].astype(jnp.int32)
    dst = dstf.reshape(NW, NCHUNK, CH)
    zeros16 = jnp.zeros((N, 16), jnp.float32)
    zeros128 = jnp.zeros((N, F), jnp.float32)
    b1r = b1.reshape(1, HID).astype(jnp.float32)
    b2r = b2.reshape(1, F).astype(jnp.float32)

    hist = _deg_call()(dstf, zeros16)
    z1 = _tc1(hist, x)
    p = _agg_call()(z1, src, dst, zeros128)
    z2 = _tc2(hist, p, z1, W1.astype(jnp.float32), b1r, W2.astype(jnp.float32))
    q = _agg_call()(z2, src, dst, zeros128)
    return _tc3(hist, q, z2, b2r)


# speedup vs baseline: 27.2388x; 1.1246x over previous
import functools

import jax
import jax.numpy as jnp
from jax import lax
from jax.experimental import pallas as pl
from jax.experimental.pallas import tpu as pltpu
from jax.experimental.pallas import tpu_sc as plsc

N = 10000
E = 320000
F = 128
HID = 256

NC = 2
NS = 16
NW = NC * NS
EPT = E // NW
CH = 80
NCHUNK = EPT // CH

RSTEP = 624
RLEN = 640


def _mesh():
    return plsc.VectorSubcoreMesh(
        core_axis_name="c", subcore_axis_name="s", num_cores=NC, num_subcores=NS
    )


def _deg_body(dst_hbm, zeros16_hbm, hist_hbm, ones_v, didx_v, hist_sh, sem):
    c = lax.axis_index("c")
    s = lax.axis_index("s")
    t = c * NS + s
    for i in range(CH):
        ones_v[i] = jnp.ones((16,), jnp.float32)
    pltpu.sync_copy(
        zeros16_hbm.at[pl.ds(s * RSTEP, RLEN)], hist_sh.at[pl.ds(s * RSTEP, RLEN)]
    )
    plsc.subcore_barrier()

    pltpu.sync_copy(dst_hbm.at[t], didx_v)

    def step(g, carry):
        pltpu.sync_copy(ones_v, hist_sh.at[didx_v.at[g]], add=True)
        return carry

    lax.fori_loop(0, NCHUNK, step, 0)

    plsc.subcore_barrier()
    pltpu.sync_copy(
        hist_sh.at[pl.ds(s * RSTEP, RLEN)], hist_hbm.at[c, pl.ds(s * RSTEP, RLEN)]
    )


@functools.cache
def _deg_call():
    return pl.kernel(
        _deg_body,
        out_type=jax.ShapeDtypeStruct((NC, N, 16), jnp.float32),
        mesh=_mesh(),
        scratch_types=[
            pltpu.VMEM((CH, 16), jnp.float32),
            pltpu.VMEM((NCHUNK, CH), jnp.int32),
            pltpu.VMEM_SHARED((N, 16), jnp.float32),
            pltpu.SemaphoreType.DMA,
        ],
    )


def _agg_body(z_hbm, src_hbm, dst_hbm, zeros_hbm, out_hbm,
              sidx_v, didx_v, rows0, rows1, acc_sh, gsem0, gsem1):
    c = lax.axis_index("c")
    s = lax.axis_index("s")
    t = c * NS + s
    pltpu.sync_copy(
        zeros_hbm.at[pl.ds(s * RSTEP, RLEN)], acc_sh.at[pl.ds(s * RSTEP, RLEN)]
    )
    pltpu.sync_copy(src_hbm.at[pl.ds(t * EPT, EPT)], sidx_v)
    pltpu.sync_copy(dst_hbm.at[t], didx_v)
    plsc.subcore_barrier()

    rows = (rows0, rows1)
    gsem = (gsem0, gsem1)

    def _wait_gather(b):
        pltpu.make_async_copy(z_hbm.at[pl.ds(0, CH)], rows[b], gsem[b]).wait()

    def _sidx(g):
        return sidx_v.at[pl.ds(g * CH, CH)]

    pltpu.async_copy(z_hbm.at[_sidx(0)], rows0, gsem0)
    pltpu.async_copy(z_hbm.at[_sidx(1)], rows1, gsem1)

    def step(k, carry):
        for b in range(2):
            g = 2 * k + b
            _wait_gather(b)
            pltpu.sync_copy(rows[b], acc_sh.at[didx_v.at[g]], add=True)
            pltpu.async_copy(z_hbm.at[_sidx(g + 2)], rows[b], gsem[b])
        return carry

    lax.fori_loop(0, NCHUNK // 2 - 1, step, 0)
    _wait_gather(0)
    pltpu.sync_copy(rows0, acc_sh.at[didx_v.at[NCHUNK - 3]], add=True)
    pltpu.async_copy(z_hbm.at[_sidx(NCHUNK - 1)], rows0, gsem0)
    _wait_gather(1)
    pltpu.sync_copy(rows1, acc_sh.at[didx_v.at[NCHUNK - 2]], add=True)
    _wait_gather(0)
    pltpu.sync_copy(rows0, acc_sh.at[didx_v.at[NCHUNK - 1]], add=True)

    plsc.subcore_barrier()
    pltpu.sync_copy(
        acc_sh.at[pl.ds(s * RSTEP, RLEN)], out_hbm.at[c, pl.ds(s * RSTEP, RLEN)]
    )


@functools.cache
def _agg_call():
    return pl.kernel(
        _agg_body,
        out_type=jax.ShapeDtypeStruct((NC, N, F), jnp.float32),
        mesh=_mesh(),
        scratch_types=[
            pltpu.VMEM((EPT,), jnp.int32),
            pltpu.VMEM((NCHUNK, CH), jnp.int32),
            pltpu.VMEM((CH, F), jnp.float32),
            pltpu.VMEM((CH, F), jnp.float32),
            pltpu.VMEM_SHARED((N, F), jnp.float32),
            pltpu.SemaphoreType.DMA,
            pltpu.SemaphoreType.DMA,
        ],
    )


BLK = 200


def _d_from_hist(hist_blk):
    deg = hist_blk[0, :, 0] + hist_blk[1, :, 0] + 1.0
    return lax.rsqrt(deg)


def _tc1_body(hist_ref, x_ref, z1_ref):
    d = _d_from_hist(hist_ref[...])
    z1_ref[...] = x_ref[...] * d[:, None]


def _tc2_body(hist_ref, p_ref, z1_ref, w1_ref, b1_ref, w2_ref, z2_ref):
    d = _d_from_hist(hist_ref[...])
    p = p_ref[...]
    agg = (p[0] + p[1] + z1_ref[...]) * d[:, None]
    h = jnp.dot(agg, w1_ref[...], preferred_element_type=jnp.float32)
    h = jnp.maximum(h + b1_ref[...], 0.0)
    g = jnp.dot(h, w2_ref[...], preferred_element_type=jnp.float32)
    z2_ref[...] = g * d[:, None]


def _tc3_body(hist_ref, q_ref, z2_ref, b2_ref, out_ref):
    d = _d_from_hist(hist_ref[...])
    q = q_ref[...]
    out_ref[...] = (q[0] + q[1] + z2_ref[...]) * d[:, None] + b2_ref[...]


_hist_spec = pl.BlockSpec((NC, BLK, 16), lambda i: (0, i, 0))
_row_spec = pl.BlockSpec((BLK, F), lambda i: (i, 0))
_pair_spec = pl.BlockSpec((NC, BLK, F), lambda i: (0, i, 0))


def _tc1(hist, x):
    return pl.pallas_call(
        _tc1_body,
        grid=(N // BLK,),
        in_specs=[_hist_spec, _row_spec],
        out_specs=_row_spec,
        out_shape=jax.ShapeDtypeStruct((N, F), jnp.float32),
    )(hist, x)


def _tc2(hist, p, z1, w1, b1, w2):
    return pl.pallas_call(
        _tc2_body,
        grid=(N // BLK,),
        in_specs=[
            _hist_spec,
            _pair_spec,
            _row_spec,
            pl.BlockSpec((F, HID), lambda i: (0, 0)),
            pl.BlockSpec((1, HID), lambda i: (0, 0)),
            pl.BlockSpec((HID, F), lambda i: (0, 0)),
        ],
        out_specs=_row_spec,
        out_shape=jax.ShapeDtypeStruct((N, F), jnp.float32),
    )(hist, p, z1, w1, b1, w2)


def _tc3(hist, q, z2, b2):
    return pl.pallas_call(
        _tc3_body,
        grid=(N // BLK,),
        in_specs=[
            _hist_spec,
            _pair_spec,
            _row_spec,
            pl.BlockSpec((1, F), lambda i: (0, 0)),
        ],
        out_specs=_row_spec,
        out_shape=jax.ShapeDtypeStruct((N, F), jnp.float32),
    )(hist, q, z2, b2)


def kernel(x, edge_index, W1, b1, W2, b2):
    x = x.astype(jnp.float32)
    src = edge_index[0].astype(jnp.int32)
    dstf = edge_index[1].astype(jnp.int32)
    dst = dstf.reshape(NW, NCHUNK, CH)
    zeros16 = jnp.zeros((N, 16), jnp.float32)
    zeros128 = jnp.zeros((N, F), jnp.float32)
    b1r = b1.reshape(1, HID).astype(jnp.float32)
    b2r = b2.reshape(1, F).astype(jnp.float32)

    hist = _deg_call()(dst, zeros16)
    z1 = _tc1(hist, x)
    p = _agg_call()(z1, src, dst, zeros128)
    z2 = _tc2(hist, p, z1, W1.astype(jnp.float32), b1r, W2.astype(jnp.float32))
    q = _agg_call()(z2, src, dst, zeros128)
    return _tc3(hist, q, z2, b2r)


# speedup vs baseline: 27.2436x; 1.0002x over previous
import functools

import jax
import jax.numpy as jnp
from jax import lax
from jax.experimental import pallas as pl
from jax.experimental.pallas import tpu as pltpu
from jax.experimental.pallas import tpu_sc as plsc

N = 10000
E = 320000
F = 128
HID = 256

NC = 2
NS = 16
NW = NC * NS
EPT = E // NW
CH = 80
NCHUNK = EPT // CH

RSTEP = 624
RLEN = 640


def _mesh():
    return plsc.VectorSubcoreMesh(
        core_axis_name="c", subcore_axis_name="s", num_cores=NC, num_subcores=NS
    )


def _deg_body(dst_hbm, zeros16_hbm, hist_hbm, ones_v, didx_v, hist_sh, sem):
    c = lax.axis_index("c")
    s = lax.axis_index("s")
    t = c * NS + s
    for i in range(CH):
        ones_v[i] = jnp.ones((16,), jnp.float32)
    pltpu.sync_copy(
        zeros16_hbm.at[pl.ds(s * RSTEP, RLEN)], hist_sh.at[pl.ds(s * RSTEP, RLEN)]
    )
    plsc.subcore_barrier()

    pltpu.sync_copy(dst_hbm.at[t], didx_v)

    def step(g, carry):
        pltpu.sync_copy(ones_v, hist_sh.at[didx_v.at[g]], add=True)
        return carry

    lax.fori_loop(0, NCHUNK, step, 0)

    plsc.subcore_barrier()
    pltpu.sync_copy(
        hist_sh.at[pl.ds(s * RSTEP, RLEN)], hist_hbm.at[c, pl.ds(s * RSTEP, RLEN)]
    )


@functools.cache
def _deg_call():
    return pl.kernel(
        _deg_body,
        out_type=jax.ShapeDtypeStruct((NC, N, 16), jnp.float32),
        mesh=_mesh(),
        scratch_types=[
            pltpu.VMEM((CH, 16), jnp.float32),
            pltpu.VMEM((NCHUNK, CH), jnp.int32),
            pltpu.VMEM_SHARED((N, 16), jnp.float32),
            pltpu.SemaphoreType.DMA,
        ],
    )


def _agg_body(z_hbm, src_hbm, dst_hbm, zeros_hbm, out_hbm,
              sidx_v, didx_v, rows0, rows1, acc_sh, gsem0, gsem1):
    c = lax.axis_index("c")
    s = lax.axis_index("s")
    t = c * NS + s
    pltpu.sync_copy(
        zeros_hbm.at[pl.ds(s * RSTEP, RLEN)], acc_sh.at[pl.ds(s * RSTEP, RLEN)]
    )
    pltpu.sync_copy(src_hbm.at[pl.ds(t * EPT, EPT)], sidx_v)
    pltpu.sync_copy(dst_hbm.at[t], didx_v)
    plsc.subcore_barrier()

    rows = (rows0, rows1)
    gsem = (gsem0, gsem1)

    def _wait_gather(b):
        pltpu.make_async_copy(z_hbm.at[pl.ds(0, CH)], rows[b], gsem[b]).wait()

    def _sidx(g):
        return sidx_v.at[pl.ds(g * CH, CH)]

    pltpu.async_copy(z_hbm.at[_sidx(0)], rows0, gsem0)
    pltpu.async_copy(z_hbm.at[_sidx(1)], rows1, gsem1)

    def step(k, carry):
        for b in range(2):
            g = 2 * k + b
            _wait_gather(b)
            pltpu.sync_copy(rows[b], acc_sh.at[didx_v.at[g]], add=True)
            pltpu.async_copy(z_hbm.at[_sidx(g + 2)], rows[b], gsem[b])
        return carry

    lax.fori_loop(0, NCHUNK // 2 - 1, step, 0)
    _wait_gather(0)
    pltpu.sync_copy(rows0, acc_sh.at[didx_v.at[NCHUNK - 3]], add=True)
    pltpu.async_copy(z_hbm.at[_sidx(NCHUNK - 1)], rows0, gsem0)
    _wait_gather(1)
    pltpu.sync_copy(rows1, acc_sh.at[didx_v.at[NCHUNK - 2]], add=True)
    _wait_gather(0)
    pltpu.sync_copy(rows0, acc_sh.at[didx_v.at[NCHUNK - 1]], add=True)

    plsc.subcore_barrier()
    pltpu.sync_copy(
        acc_sh.at[pl.ds(s * RSTEP, RLEN)], out_hbm.at[c, pl.ds(s * RSTEP, RLEN)]
    )


@functools.cache
def _agg_call():
    return pl.kernel(
        _agg_body,
        out_type=jax.ShapeDtypeStruct((NC, N, F), jnp.float32),
        mesh=_mesh(),
        scratch_types=[
            pltpu.VMEM((EPT,), jnp.int32),
            pltpu.VMEM((NCHUNK, CH), jnp.int32),
            pltpu.VMEM((CH, F), jnp.float32),
            pltpu.VMEM((CH, F), jnp.float32),
            pltpu.VMEM_SHARED((N, F), jnp.float32),
            pltpu.SemaphoreType.DMA,
            pltpu.SemaphoreType.DMA,
        ],
    )


BLK = 200


def _d_from_hist(hist_blk):
    deg = hist_blk[0, :, 0] + hist_blk[1, :, 0] + 1.0
    return lax.rsqrt(deg)


def _tc1_body(hist_ref, x_ref, z1_ref):
    d = _d_from_hist(hist_ref[...])
    z1_ref[...] = x_ref[...] * d[:, None]


def _tc2_body(hist_ref, p_ref, z1_ref, w1_ref, b1_ref, w2_ref, z2_ref):
    d = _d_from_hist(hist_ref[...])
    p = p_ref[...]
    agg = (p[0] + p[1] + z1_ref[...]) * d[:, None]
    h = jnp.dot(agg, w1_ref[...], preferred_element_type=jnp.float32)
    h = jnp.maximum(h + b1_ref[...], 0.0)
    g = jnp.dot(h, w2_ref[...], preferred_element_type=jnp.float32)
    z2_ref[...] = g * d[:, None]


def _tc3_body(hist_ref, q_ref, z2_ref, b2_ref, out_ref):
    d = _d_from_hist(hist_ref[...])
    q = q_ref[...]
    out_ref[...] = (q[0] + q[1] + z2_ref[...]) * d[:, None] + b2_ref[...]


_hist_spec = pl.BlockSpec((NC, BLK, 16), lambda i: (0, i, 0))
_row_spec = pl.BlockSpec((BLK, F), lambda i: (i, 0))
_pair_spec = pl.BlockSpec((NC, BLK, F), lambda i: (0, i, 0))


def _tc1(hist, x):
    return pl.pallas_call(
        _tc1_body,
        grid=(N // BLK,),
        in_specs=[_hist_spec, _row_spec],
        out_specs=_row_spec,
        out_shape=jax.ShapeDtypeStruct((N, F), jnp.float32),
    )(hist, x)


def _tc2(hist, p, z1, w1, b1, w2):
    return pl.pallas_call(
        _tc2_body,
        grid=(N // BLK,),
        in_specs=[
            _hist_spec,
            _pair_spec,
            _row_spec,
            pl.BlockSpec((F, HID), lambda i: (0, 0)),
            pl.BlockSpec((1, HID), lambda i: (0, 0)),
            pl.BlockSpec((HID, F), lambda i: (0, 0)),
        ],
        out_specs=_row_spec,
        out_shape=jax.ShapeDtypeStruct((N, F), jnp.float32),
    )(hist, p, z1, w1, b1, w2)


def _tc3(hist, q, z2, b2):
    return pl.pallas_call(
        _tc3_body,
        grid=(N // BLK,),
        in_specs=[
            _hist_spec,
            _pair_spec,
            _row_spec,
            pl.BlockSpec((1, F), lambda i: (0, 0)),
        ],
        out_specs=_row_spec,
        out_shape=jax.ShapeDtypeStruct((N, F), jnp.float32),
    )(hist, q, z2, b2)


def kernel(x, edge_index, W1, b1, W2, b2):
    x = x.astype(jnp.float32)
    src = edge_index[0].astype(jnp.int32)
    dst = edge_index[1].astype(jnp.int32).reshape(NW, NCHUNK, CH)
    zeros16 = jnp.zeros((N, 16), jnp.float32)
    zeros128 = jnp.zeros((N, F), jnp.float32)
    b1r = b1.reshape(1, HID).astype(jnp.float32)
    b2r = b2.reshape(1, F).astype(jnp.float32)

    hist = _deg_call()(dst, zeros16)
    z1 = _tc1(hist, x)
    p = _agg_call()(z1, src, dst, zeros128)
    z2 = _tc2(hist, p, z1, W1.astype(jnp.float32), b1r, W2.astype(jnp.float32))
    q = _agg_call()(z2, src, dst, zeros128)
    return _tc3(hist, q, z2, b2r)
